# baseline (speedup 1.0000x reference)
import functools

import jax
import jax.numpy as jnp
from jax import lax
from jax.experimental import pallas as pl
from jax.experimental.pallas import tpu as pltpu
from jax.experimental.pallas import tpu_sc as plsc

B = 64
N = 32768
L = 16
NC, NS = 2, 16
QCOLS = N // 4
NCH = 4
CCOLS = QCOLS // NCH
ITERS = CCOLS // L
BIG_IDX = 1 << 30

_mesh = plsc.VectorSubcoreMesh(core_axis_name="c", subcore_axis_name="s")


@functools.partial(
    pl.kernel,
    out_type=(jax.ShapeDtypeStruct((4 * 128,), jnp.float32),
              jax.ShapeDtypeStruct((2 * 128,), jnp.float32),
              jax.ShapeDtypeStruct((2 * 128,), jnp.float32),
              jax.ShapeDtypeStruct((2 * 16 * 32,), jnp.float32)),
    mesh=_mesh,
    scratch_types=[
        pltpu.VMEM((8, CCOLS), jnp.float32),
        pltpu.VMEM((8, CCOLS), jnp.float32),
        pltpu.VMEM((16,), jnp.float32),
        pltpu.VMEM((16,), jnp.float32),
        pltpu.VMEM((128,), jnp.float32),
        pltpu.VMEM((16, 8, 128), jnp.float32),
        pltpu.VMEM((8, 2, 128), jnp.float32),
        pltpu.VMEM((16,), jnp.float32),
        pltpu.SemaphoreType.DMA,
        pltpu.SemaphoreType.DMA,
        pltpu.SemaphoreType.DMA,
    ],
    compiler_params=pltpu.CompilerParams(needs_layout_passes=False),
)
def _select_kernel(ep_hbm, line_hbm, perp_hbm,
                   oe_hbm, ol_hbm, op_hbm, xpk_hbm,
                   c0, c1, valbuf, idxbuf, qpk,
                   tbuf, lpbuf, obuf,
                   sem0, sem1, sem_f):
    cid = lax.axis_index("c")
    sid = lax.axis_index("s")
    band = cid * 4 + (sid >> 2)
    k = sid & 3
    rb = pl.multiple_of(band * 8, 8)
    col0 = pl.multiple_of(k * QCOLS, 128)

    iota = lax.iota(jnp.int32, L)

    def block_scan(buf, cand_base, accs):
        pos0 = cand_base + iota

        def body(i, accs):
            pos = pos0 + i * L
            out = []
            for s_r in range(8):
                mx, ai = accs[s_r]
                cv = buf[s_r, pl.ds(i * L, L)]
                m = cv > mx
                out.append((jnp.where(m, cv, mx), jnp.where(m, pos, ai)))
            return tuple(out)

        return lax.fori_loop(0, ITERS, body, tuple(accs), unroll=2)

    bufs = (c0, c1)
    sems = (sem0, sem1)
    hs = [None, None]
    for ci in range(2):
        hs[ci] = pltpu.async_copy(
            ep_hbm.at[0, pl.ds(rb, 8), pl.ds(col0 + ci * CCOLS, CCOLS)],
            bufs[ci], sems[ci])

    neg_inf = jnp.full((L,), -jnp.inf, jnp.float32)
    zeros_i = jnp.zeros((L,), jnp.int32)
    accs = [(neg_inf, zeros_i)] * 8

    for ci in range(NCH):
        par = ci & 1
        hs[par].wait()
        accs = block_scan(bufs[par], col0 + ci * CCOLS, accs)
        if ci + 2 < NCH:
            hs[par] = pltpu.async_copy(
                ep_hbm.at[0, pl.ds(rb, 8),
                          pl.ds(col0 + (ci + 2) * CCOLS, CCOLS)],
                bufs[par], sems[par])

    valv = jnp.zeros((L,), jnp.float32)
    idxv = jnp.zeros((L,), jnp.int32)
    for s_r in range(8):
        mx, ai = accs[s_r]
        row_max = jnp.max(mx)
        g = jnp.min(jnp.where(mx == row_max, ai, jnp.int32(BIG_IDX)))
        valv = jnp.where(iota == s_r, row_max, valv)
        idxv = jnp.where(iota == s_r, g, idxv)
    valbuf[...] = valv
    idxbuf[...] = plsc.bitcast(idxv, jnp.float32)

    slot = cid * 16 + sid
    pltpu.sync_copy(valbuf,
                    xpk_hbm.at[pl.ds(pl.multiple_of(slot * 32, 8), 16)])
    pltpu.sync_copy(idxbuf,
                    xpk_hbm.at[pl.ds(pl.multiple_of(slot * 32 + 16, 8), 16)])
    plsc.subcore_barrier()
    bslot = cid * 16 + (sid & ~3)
    pltpu.sync_copy(xpk_hbm.at[pl.ds(pl.multiple_of(bslot * 32, 8), 128)],
                    qpk)

    mv = qpk[pl.ds(0, L)]
    mi = plsc.bitcast(qpk[pl.ds(16, L)], jnp.int32)
    for q in range(1, 4):
        v = qpk[pl.ds(q * 32, L)]
        i_ = plsc.bitcast(qpk[pl.ds(q * 32 + 16, L)], jnp.int32)
        m = v > mv
        mv = jnp.where(m, v, mv)
        mi = jnp.where(m, i_, mi)

    mi = jnp.clip(mi, 0, N - 1)
    glv = mi & 127
    gbases = []
    for l in range(8):
        g_l = jnp.max(jnp.where(iota == l, mi, jnp.int32(0)))
        gbases.append(pl.multiple_of((g_l >> 7) << 7, 128))

    @pl.when(k < 2)
    def _():
        cbase = 1 + k * 2
        hs = []
        for dc in range(2):
            for l in range(8):
                hs.append(pltpu.async_copy(
                    ep_hbm.at[cbase + dc, pl.ds(rb, 8), pl.ds(gbases[l], 128)],
                    tbuf.at[dc * 8 + l], sem_f))
        for h in hs:
            h.wait()
        for dc in range(2):
            vals = plsc.load_gather(
                tbuf, [jnp.int32(dc * 8) + (iota & 7), iota & 7, glv])
            obuf[...] = vals
            comp = cbase + dc - 1
            pltpu.sync_copy(
                obuf.at[pl.ds(0, 8)],
                oe_hbm.at[pl.ds(pl.multiple_of(comp * 128 + rb, 8), 8)])

    for kk, (src_hbm, dst_hbm) in ((2, (line_hbm, ol_hbm)),
                                   (3, (perp_hbm, op_hbm))):
        @pl.when(k == kk)
        def _(src_hbm=src_hbm, dst_hbm=dst_hbm):
            hs = []
            for l in range(8):
                hs.append(pltpu.async_copy(
                    src_hbm.at[rb + l, pl.ds(0, 2), pl.ds(gbases[l], 128)],
                    lpbuf.at[l], sem_f))
            for h in hs:
                h.wait()
            for j in range(2):
                vals = plsc.load_gather(
                    lpbuf, [iota & 7, jnp.full((L,), j, jnp.int32), glv])
                obuf[...] = vals
                pltpu.sync_copy(
                    obuf.at[pl.ds(0, 8)],
                    dst_hbm.at[pl.ds(pl.multiple_of(j * 128 + rb, 8), 8)])


def kernel(endpoints_pred, line_vec, perp_vec):
    ep_t = jnp.transpose(endpoints_pred, (2, 0, 1))
    line_t = jnp.transpose(line_vec, (0, 2, 1))
    perp_t = jnp.transpose(perp_vec, (0, 2, 1))
    oe, ol, op = _select_kernel(ep_t, line_t, perp_t)[:3]
    selected_endpoints = jnp.transpose(oe.reshape(2, 2, 128), (2, 0, 1))[:B]
    selected_line_vec = jnp.transpose(ol.reshape(2, 128), (1, 0))[:B]
    selected_perp_vec = jnp.transpose(op.reshape(2, 128), (1, 0))[:B]
    return (selected_endpoints, selected_line_vec, selected_perp_vec)
